# baseline (speedup 1.0000x reference)
import functools

import jax
import jax.numpy as jnp
from jax import lax
from jax.experimental import pallas as pl
from jax.experimental.pallas import tpu as pltpu
from jax.experimental.pallas import tpu_sc as plsc

_B = 4
_CHUNK_ROWS = 32


def kernel(batch_size, table):
    n, f = table.shape
    info = plsc.get_sparse_core_info()
    nw = info.num_cores * info.num_subcores
    rows_per_w = n // nw
    n_chunks = rows_per_w // _CHUNK_ROWS

    mesh = plsc.VectorSubcoreMesh(core_axis_name="c", subcore_axis_name="s")

    @functools.partial(
        pl.kernel,
        mesh=mesh,
        out_type=jax.ShapeDtypeStruct((_B, n, f), jnp.float32),
        scratch_types=[
            pltpu.VMEM((2, _CHUNK_ROWS, f), jnp.float32),
            pltpu.SemaphoreType.DMA,
            pltpu.SemaphoreType.DMA,
            pltpu.SemaphoreType.DMA,
            pltpu.SemaphoreType.DMA,
        ],
    )
    def k(table_hbm, out_hbm, bufs, in_sem0, in_sem1, out_sem0, out_sem1):
        wid = lax.axis_index("s") * info.num_cores + lax.axis_index("c")
        base = wid * rows_per_w
        in_sems = (in_sem0, in_sem1)
        out_sems = (out_sem0, out_sem1)

        def stage(c):
            r0 = base + c * _CHUNK_ROWS
            return pltpu.async_copy(
                table_hbm.at[pl.ds(r0, _CHUNK_ROWS)], bufs.at[c % 2], in_sems[c % 2]
            )

        in_flight = stage(0)
        out_flight = [None, None]
        for c in range(n_chunks):
            cur = c % 2
            if c + 1 < n_chunks:
                nxt = (c + 1) % 2
                if out_flight[nxt] is not None:
                    for h in out_flight[nxt]:
                        h.wait()
                next_in = stage(c + 1)
            in_flight.wait()
            r0 = base + c * _CHUNK_ROWS
            out_flight[cur] = [
                pltpu.async_copy(
                    bufs.at[cur], out_hbm.at[b, pl.ds(r0, _CHUNK_ROWS)], out_sems[cur]
                )
                for b in range(_B)
            ]
            if c + 1 < n_chunks:
                in_flight = next_in
        for hs in out_flight:
            if hs is not None:
                for h in hs:
                    h.wait()

    return k(table)


# speedup vs baseline: 1.2479x; 1.2479x over previous
import jax
import jax.numpy as jnp
from jax.experimental import pallas as pl
from jax.experimental.pallas import tpu as pltpu

_B = 4
_ROWS_PER_BLOCK = 512


def _body(t_hbm, o_hbm, buf, in_sem0, in_sem1, out_sem0, out_sem1):
    n = t_hbm.shape[0]
    n_blocks = n // _ROWS_PER_BLOCK
    in_sems = (in_sem0, in_sem1)
    out_sems = (out_sem0, out_sem1)

    def stage(c):
        r0 = c * _ROWS_PER_BLOCK
        cp = pltpu.make_async_copy(
            t_hbm.at[pl.ds(r0, _ROWS_PER_BLOCK)], buf.at[c % 2], in_sems[c % 2]
        )
        cp.start()
        return cp

    in_flight = stage(0)
    out_flight = [None, None]
    for c in range(n_blocks):
        cur = c % 2
        if c + 1 < n_blocks:
            nxt = (c + 1) % 2
            if out_flight[nxt] is not None:
                for h in out_flight[nxt]:
                    h.wait()
            next_in = stage(c + 1)
        in_flight.wait()
        r0 = c * _ROWS_PER_BLOCK
        outs = []
        for b in range(_B):
            cp = pltpu.make_async_copy(
                buf.at[cur], o_hbm.at[b, pl.ds(r0, _ROWS_PER_BLOCK)], out_sems[cur]
            )
            cp.start()
            outs.append(cp)
        out_flight[cur] = outs
        if c + 1 < n_blocks:
            in_flight = next_in
    for hs in out_flight:
        if hs is not None:
            for h in hs:
                h.wait()


def kernel(batch_size, table):
    n, f = table.shape
    out = pl.pallas_call(
        _body,
        in_specs=[pl.BlockSpec(memory_space=pl.ANY)],
        out_specs=pl.BlockSpec(memory_space=pl.ANY),
        out_shape=jax.ShapeDtypeStruct((_B, n, f), jnp.float32),
        scratch_shapes=[
            pltpu.VMEM((2, _ROWS_PER_BLOCK, f), jnp.float32),
            pltpu.SemaphoreType.DMA,
            pltpu.SemaphoreType.DMA,
            pltpu.SemaphoreType.DMA,
            pltpu.SemaphoreType.DMA,
        ],
    )(table)
    return out


# speedup vs baseline: 1.3375x; 1.0718x over previous
import jax
import jax.numpy as jnp
from jax.experimental import pallas as pl

_B = 4
_ROWS_PER_BLOCK = 256


def _body(t_ref, o_ref):
    x = t_ref[...]
    o_ref[...] = jnp.broadcast_to(x[None], (_B,) + x.shape)


def kernel(batch_size, table):
    n, f = table.shape
    r = _ROWS_PER_BLOCK
    out = pl.pallas_call(
        _body,
        grid=(n // r,),
        in_specs=[pl.BlockSpec((r, f), lambda i: (i, 0))],
        out_specs=pl.BlockSpec((_B, r, f), lambda i: (0, i, 0)),
        out_shape=jax.ShapeDtypeStruct((_B, n, f), jnp.float32),
    )(table)
    return out


# speedup vs baseline: 1.4563x; 1.0888x over previous
import jax
import jax.numpy as jnp
from jax.experimental import pallas as pl

_B = 4
_ROWS_PER_BLOCK = 1024


def _body(t_ref, o_ref):
    x = t_ref[...]
    o_ref[...] = jnp.broadcast_to(x[None], (_B,) + x.shape)


def kernel(batch_size, table):
    n, f = table.shape
    r = _ROWS_PER_BLOCK
    out = pl.pallas_call(
        _body,
        grid=(n // r,),
        in_specs=[pl.BlockSpec((r, f), lambda i: (i, 0))],
        out_specs=pl.BlockSpec((_B, r, f), lambda i: (0, i, 0)),
        out_shape=jax.ShapeDtypeStruct((_B, n, f), jnp.float32),
    )(table)
    return out
